# baseline (speedup 1.0000x reference)
import functools

import jax
import jax.numpy as jnp
from jax.experimental import pallas as pl

DIM = 32
NN = 16
BK = 32


def _agg_kernel(v0_ref, v1_ref, rv0_ref, rv1_ref, v2_ref, w_ref, b_ref, out_ref):
    bk = v0_ref.shape[0]
    x = bk * NN
    W = w_ref[...]
    b = b_ref[...]

    v1 = v1_ref[...]
    rv1 = rv1_ref[...]
    v2 = v2_ref[...]

    s1 = jnp.sum(v1[:, None, :] * rv1, axis=-1)
    s1 = s1 - jnp.max(s1, axis=-1, keepdims=True)
    e1 = jnp.exp(s1)
    w1 = e1 / jnp.sum(e1, axis=-1, keepdims=True)
    agg1 = jnp.sum(w1[:, :, None] * v2, axis=1)
    h1 = jax.nn.sigmoid((v1 + agg1) @ W + b)

    v0 = v0_ref[...]
    rv0 = rv0_ref[...]
    v1g = v1.reshape(bk, NN, DIM)
    s0 = jnp.sum(v0[:, None, :] * rv0, axis=-1)
    s0 = s0 - jnp.max(s0, axis=-1, keepdims=True)
    e0 = jnp.exp(s0)
    w0 = e0 / jnp.sum(e0, axis=-1, keepdims=True)
    agg0 = jnp.sum(w0[:, :, None] * v1g, axis=1)
    h0 = jax.nn.sigmoid((v0 + agg0) @ W + b)

    sh = jnp.sum(h0[:, None, :] * rv0, axis=-1)
    sh = sh - jnp.max(sh, axis=-1, keepdims=True)
    eh = jnp.exp(sh)
    wh = eh / jnp.sum(eh, axis=-1, keepdims=True)
    h1g = h1.reshape(bk, NN, DIM)
    aggh = jnp.sum(wh[:, :, None] * h1g, axis=1)
    out_ref[...] = jnp.tanh((h0 + aggh) @ W + b)


def kernel(drug_entity_list, adj_ent, adj_rel, drug_table, ent_table, rel_table, W, b):
    B = drug_entity_list.shape[0]
    u = drug_entity_list
    e1 = jnp.take(adj_ent, u, axis=0)
    r0 = jnp.take(adj_rel, u, axis=0)
    e1f = e1.reshape(-1)
    e2 = jnp.take(adj_ent, e1f, axis=0)
    r1 = jnp.take(adj_rel, e1f, axis=0)
    v0 = jnp.take(ent_table, u, axis=0)
    v1 = jnp.take(ent_table, e1f, axis=0)
    v2 = jnp.take(ent_table, e2.reshape(-1), axis=0).reshape(B * NN, NN, DIM)
    rv0 = jnp.take(rel_table, r0.reshape(-1), axis=0).reshape(B, NN, DIM)
    rv1 = jnp.take(rel_table, r1.reshape(-1), axis=0).reshape(B * NN, NN, DIM)

    grid = B // BK
    out = pl.pallas_call(
        _agg_kernel,
        grid=(grid,),
        in_specs=[
            pl.BlockSpec((BK, DIM), lambda i: (i, 0)),
            pl.BlockSpec((BK * NN, DIM), lambda i: (i, 0)),
            pl.BlockSpec((BK, NN, DIM), lambda i: (i, 0, 0)),
            pl.BlockSpec((BK * NN, NN, DIM), lambda i: (i, 0, 0)),
            pl.BlockSpec((BK * NN, NN, DIM), lambda i: (i, 0, 0)),
            pl.BlockSpec((DIM, DIM), lambda i: (0, 0)),
            pl.BlockSpec((1, DIM), lambda i: (0, 0)),
        ],
        out_specs=pl.BlockSpec((BK, DIM), lambda i: (i, 0)),
        out_shape=jax.ShapeDtypeStruct((B, DIM), jnp.float32),
    )(v0, v1, rv0, rv1, v2, W, b.reshape(1, DIM))
    return out


# speedup vs baseline: 1.9246x; 1.9246x over previous
import functools

import jax
import jax.numpy as jnp
from jax.experimental import pallas as pl

DIM = 32
NN = 16
BK = 32


def _agg_kernel(v0_ref, v1_ref, rv0_ref, s1_ref, v2_ref, w_ref, b_ref, out_ref):
    bk = v0_ref.shape[0]
    W = w_ref[...]
    b = b_ref[...]

    v1 = v1_ref[...]
    v2 = v2_ref[...]

    s1 = s1_ref[...]
    s1 = s1 - jnp.max(s1, axis=-1, keepdims=True)
    e1 = jnp.exp(s1)
    w1 = e1 / jnp.sum(e1, axis=-1, keepdims=True)
    agg1 = jnp.sum(w1[:, :, None] * v2, axis=1)
    h1 = jax.nn.sigmoid((v1 + agg1) @ W + b)

    v0 = v0_ref[...]
    rv0 = rv0_ref[...]
    v1g = v1.reshape(bk, NN, DIM)
    s0 = jnp.sum(v0[:, None, :] * rv0, axis=-1)
    s0 = s0 - jnp.max(s0, axis=-1, keepdims=True)
    e0 = jnp.exp(s0)
    w0 = e0 / jnp.sum(e0, axis=-1, keepdims=True)
    agg0 = jnp.sum(w0[:, :, None] * v1g, axis=1)
    h0 = jax.nn.sigmoid((v0 + agg0) @ W + b)

    sh = jnp.sum(h0[:, None, :] * rv0, axis=-1)
    sh = sh - jnp.max(sh, axis=-1, keepdims=True)
    eh = jnp.exp(sh)
    wh = eh / jnp.sum(eh, axis=-1, keepdims=True)
    h1g = h1.reshape(bk, NN, DIM)
    aggh = jnp.sum(wh[:, :, None] * h1g, axis=1)
    out_ref[...] = jnp.tanh((h0 + aggh) @ W + b)


def kernel(drug_entity_list, adj_ent, adj_rel, drug_table, ent_table, rel_table, W, b):
    B = drug_entity_list.shape[0]
    u = drug_entity_list
    e1 = jnp.take(adj_ent, u, axis=0)
    r0 = jnp.take(adj_rel, u, axis=0)
    e1f = e1.reshape(-1)
    e2 = jnp.take(adj_ent, e1f, axis=0)
    r1 = jnp.take(adj_rel, e1f, axis=0)
    v0 = jnp.take(ent_table, u, axis=0)
    v1 = jnp.take(ent_table, e1f, axis=0)
    v2 = jnp.take(ent_table, e2.reshape(-1), axis=0).reshape(B * NN, NN, DIM)
    rv0 = jnp.take(rel_table, r0.reshape(-1), axis=0).reshape(B, NN, DIM)
    sp1 = v1 @ rel_table.T
    s1pre = jnp.take_along_axis(sp1, r1, axis=-1)

    grid = B // BK
    out = pl.pallas_call(
        _agg_kernel,
        grid=(grid,),
        in_specs=[
            pl.BlockSpec((BK, DIM), lambda i: (i, 0)),
            pl.BlockSpec((BK * NN, DIM), lambda i: (i, 0)),
            pl.BlockSpec((BK, NN, DIM), lambda i: (i, 0, 0)),
            pl.BlockSpec((BK * NN, NN), lambda i: (i, 0)),
            pl.BlockSpec((BK * NN, NN, DIM), lambda i: (i, 0, 0)),
            pl.BlockSpec((DIM, DIM), lambda i: (0, 0)),
            pl.BlockSpec((1, DIM), lambda i: (0, 0)),
        ],
        out_specs=pl.BlockSpec((BK, DIM), lambda i: (i, 0)),
        out_shape=jax.ShapeDtypeStruct((B, DIM), jnp.float32),
    )(v0, v1, rv0, s1pre, v2, W, b.reshape(1, DIM))
    return out


# speedup vs baseline: 17.4328x; 9.0577x over previous
import functools

import jax
import jax.numpy as jnp
from jax import lax
from jax.experimental import pallas as pl
from jax.experimental.pallas import tpu as pltpu
from jax.experimental.pallas import tpu_sc as plsc

DIM = 32
NN = 16
NREL = 64
NC = 2
NS = 16
NW = NC * NS
CH = 4
L = 16


def _bc(x, dtype=jnp.float32):
    return lax.broadcast(x, (L,))


def _sc_body(u2d, adj_ent, adj_rel, ent_table, relT, pre0, pre1, r0o,
             u_c, e1c, r0c, v0c, e1f, e2c, r1c, v1c, e2fs, v2c,
             pre0c, pre1c, relT_s, sem0, sem1, sem2):
    wid = lax.axis_index("s") * NC + lax.axis_index("c")
    qw = u2d.shape[0] * CH // NW
    nch = qw // CH

    pltpu.sync_copy(relT, relT_s)
    pltpu.sync_copy(u2d.at[pl.ds(wid * nch, nch)], u_c)

    def chunk(ci, carry):
        q0 = wid * qw + ci * CH

        idx0 = u_c.at[ci]
        c1 = pltpu.async_copy(adj_ent.at[idx0], e1c, sem0)
        c2 = pltpu.async_copy(adj_rel.at[idx0], r0c, sem0)
        c3 = pltpu.async_copy(ent_table.at[idx0], v0c, sem0)
        c1.wait(); c2.wait(); c3.wait()

        for q in range(CH):
            e1f[pl.ds(q * NN, NN)] = e1c[q]

        c4 = pltpu.async_copy(adj_ent.at[e1f], e2c, sem1)
        c5 = pltpu.async_copy(adj_rel.at[e1f], r1c, sem1)
        c6 = pltpu.async_copy(ent_table.at[e1f], v1c, sem1)
        c4.wait(); c5.wait(); c6.wait()

        for k in range(CH * NN):
            e2fs[k // 8][pl.ds((k % 8) * NN, NN)] = e2c[k]

        nbatch = CH * NN * NN // 128
        cps = [pltpu.async_copy(ent_table.at[e2fs[k]],
                                v2c.at[pl.ds(k * 128, 128)], sem2)
               for k in range(nbatch)]
        for cp in cps:
            cp.wait()

        def jbody(row, carry2):
            r1row = r1c[row]
            v1a = v1c[row, 0:L]
            v1b = v1c[row, L:DIM]
            s = jnp.zeros((L,), jnp.float32)
            for d in range(L):
                rl = plsc.load_gather(relT_s, [r1row + d * NREL])
                s = s + rl * _bc(v1a[d])
            for d in range(L):
                rl = plsc.load_gather(relT_s, [r1row + (L + d) * NREL])
                s = s + rl * _bc(v1b[d])
            m = jnp.max(s)
            e = jnp.exp(s - _bc(m))
            ssum = jnp.sum(e)
            acc0 = jnp.zeros((L,), jnp.float32)
            acc1 = jnp.zeros((L,), jnp.float32)
            base = row * NN
            for n in range(NN):
                wn = _bc(e[n])
                acc0 = acc0 + wn * v2c[base + n, 0:L]
                acc1 = acc1 + wn * v2c[base + n, L:DIM]
            inv = jnp.ones((L,), jnp.float32) / _bc(ssum)
            pre1c[row, 0:L] = v1a + acc0 * inv
            pre1c[row, L:DIM] = v1b + acc1 * inv
            return carry2

        lax.fori_loop(0, CH * NN, jbody, 0)

        for q in range(CH):
            r0row = r0c[q]
            v0a = v0c[q, 0:L]
            v0b = v0c[q, L:DIM]
            s = jnp.zeros((L,), jnp.float32)
            for d in range(L):
                rl = plsc.load_gather(relT_s, [r0row + d * NREL])
                s = s + rl * _bc(v0a[d])
            for d in range(L):
                rl = plsc.load_gather(relT_s, [r0row + (L + d) * NREL])
                s = s + rl * _bc(v0b[d])
            m = jnp.max(s)
            e = jnp.exp(s - _bc(m))
            ssum = jnp.sum(e)
            acc0 = jnp.zeros((L,), jnp.float32)
            acc1 = jnp.zeros((L,), jnp.float32)
            for n in range(NN):
                wn = _bc(e[n])
                acc0 = acc0 + wn * v1c[q * NN + n, 0:L]
                acc1 = acc1 + wn * v1c[q * NN + n, L:DIM]
            inv = jnp.ones((L,), jnp.float32) / _bc(ssum)
            pre0c[q, 0:L] = v0a + acc0 * inv
            pre0c[q, L:DIM] = v0b + acc1 * inv

        pltpu.sync_copy(pre0c, pre0.at[pl.ds(q0, CH)])
        pltpu.sync_copy(pre1c, pre1.at[pl.ds(q0 * NN, CH * NN)])
        pltpu.sync_copy(r0c, r0o.at[pl.ds(q0, CH)])
        return carry

    lax.fori_loop(0, nch, chunk, 0)


def _sc_call(u, adj_ent, adj_rel, ent_table, rel_table):
    B = u.shape[0]
    mesh = plsc.VectorSubcoreMesh(core_axis_name="c", subcore_axis_name="s",
                                  num_cores=NC, num_subcores=NS)
    qw = B // NW
    nch = qw // CH
    body = functools.partial(
        pl.kernel,
        out_type=(
            jax.ShapeDtypeStruct((B, DIM), jnp.float32),
            jax.ShapeDtypeStruct((B * NN, DIM), jnp.float32),
            jax.ShapeDtypeStruct((B, NN), jnp.int32),
        ),
        mesh=mesh,
        compiler_params=pltpu.CompilerParams(needs_layout_passes=False,
                                             use_tc_tiling_on_sc=False),
        scratch_types=[
            pltpu.VMEM((nch, CH), jnp.int32),
            pltpu.VMEM((CH, NN), jnp.int32),
            pltpu.VMEM((CH, NN), jnp.int32),
            pltpu.VMEM((CH, DIM), jnp.float32),
            pltpu.VMEM((CH * NN,), jnp.int32),
            pltpu.VMEM((CH * NN, NN), jnp.int32),
            pltpu.VMEM((CH * NN, NN), jnp.int32),
            pltpu.VMEM((CH * NN, DIM), jnp.float32),
            [pltpu.VMEM((128,), jnp.int32)
             for _ in range(CH * NN * NN // 128)],
            pltpu.VMEM((CH * NN * NN, DIM), jnp.float32),
            pltpu.VMEM((CH, DIM), jnp.float32),
            pltpu.VMEM((CH * NN, DIM), jnp.float32),
            pltpu.VMEM((DIM * NREL,), jnp.float32),
            pltpu.SemaphoreType.DMA,
            pltpu.SemaphoreType.DMA,
            pltpu.SemaphoreType.DMA,
        ],
    )(_sc_body)
    u2d = u.reshape(B // CH, CH)
    relT = rel_table.T.reshape(-1)
    return body(u2d, adj_ent, adj_rel, ent_table, relT)


BK2 = 256


def _tail_kernel(pre0_ref, pre1_ref, r0_ref, relT_ref, w_ref, b_ref, out_ref):
    bk = pre0_ref.shape[0]
    W = w_ref[...]
    b = b_ref[...]
    h0 = jax.nn.sigmoid(pre0_ref[...] @ W + b)
    h1 = jax.nn.sigmoid(pre1_ref[...] @ W + b)
    sp = h0 @ relT_ref[...]
    r0 = r0_ref[...]
    rid = lax.broadcasted_iota(jnp.int32, (bk, NN, NREL), 2)
    onehot = (r0[:, :, None] == rid).astype(jnp.float32)
    sh = jnp.sum(onehot * sp[:, None, :], axis=-1)
    sh = sh - jnp.max(sh, axis=-1, keepdims=True)
    eh = jnp.exp(sh)
    wh = eh / jnp.sum(eh, axis=-1, keepdims=True)
    h1g = h1.reshape(bk, NN, DIM)
    aggh = jnp.sum(wh[:, :, None] * h1g, axis=1)
    out_ref[...] = jnp.tanh((h0 + aggh) @ W + b)


def kernel(drug_entity_list, adj_ent, adj_rel, drug_table, ent_table, rel_table, W, b):
    B = drug_entity_list.shape[0]
    pre0, pre1, r0 = _sc_call(drug_entity_list, adj_ent, adj_rel, ent_table,
                              rel_table)
    grid = B // BK2
    out = pl.pallas_call(
        _tail_kernel,
        grid=(grid,),
        in_specs=[
            pl.BlockSpec((BK2, DIM), lambda i: (i, 0)),
            pl.BlockSpec((BK2 * NN, DIM), lambda i: (i, 0)),
            pl.BlockSpec((BK2, NN), lambda i: (i, 0)),
            pl.BlockSpec((DIM, NREL), lambda i: (0, 0)),
            pl.BlockSpec((DIM, DIM), lambda i: (0, 0)),
            pl.BlockSpec((1, DIM), lambda i: (0, 0)),
        ],
        out_specs=pl.BlockSpec((BK2, DIM), lambda i: (i, 0)),
        out_shape=jax.ShapeDtypeStruct((B, DIM), jnp.float32),
    )(pre0, pre1, r0, rel_table.T, W, b.reshape(1, DIM))
    return out
